# baseline (speedup 1.0000x reference)
import functools

import jax
import jax.numpy as jnp
from jax import lax
from jax.experimental import pallas as pl
from jax.experimental.pallas import tpu as pltpu
from jax.experimental.pallas import tpu_sc as plsc

N_NODES = 10000
N_EDGES = 320000
D = 128

NC = 2
NS = 16
NW = NC * NS
CS = 120
SLOW_C = 0
SLOW_CPW = 60
FAST_CPW = 108
NGB = 3
NIB = 6
TOT_CHUNKS = NS * (SLOW_CPW + FAST_CPW)
E_PAD = TOT_CHUNKS * CS
N_PAD = 10112
STRIPE = N_PAD // NS


def _sc_aggregate(x_pad, idxp, zrows):
    mesh = plsc.VectorSubcoreMesh(core_axis_name="c", subcore_axis_name="s",
                                  num_cores=NC, num_subcores=NS)

    @functools.partial(
        pl.kernel,
        out_type=jax.ShapeDtypeStruct((NC, N_PAD, D), jnp.float32),
        mesh=mesh,
        scratch_types=(
            [pltpu.VMEM((2, CS), jnp.int32) for _ in range(NIB)]
            + [pltpu.VMEM((CS, D), jnp.float32) for _ in range(NGB)]
            + [pltpu.SemaphoreType.DMA] * (NIB + 2 * NGB)
            + [pltpu.VMEM_SHARED((N_PAD, D), jnp.float32)]
        ),
    )
    def agg_kernel(x_hbm, idx_hbm, z_hbm, out_hbm, *scr):
        ibufs = scr[:NIB]
        gbufs = scr[NIB:NIB + NGB]
        isems = scr[NIB + NGB:2 * NIB + NGB]
        gsems = scr[2 * NIB + NGB:2 * NIB + 2 * NGB]
        ssems = scr[2 * NIB + 2 * NGB:2 * NIB + 3 * NGB]
        acc = scr[2 * NIB + 3 * NGB]
        c = lax.axis_index("c")
        s = lax.axis_index("s")
        base = s * STRIPE
        is_slow = c == SLOW_C
        cbase = jnp.where(is_slow, s * SLOW_CPW,
                          NS * SLOW_CPW + s * FAST_CPW)
        n = jnp.where(is_slow, SLOW_CPW, FAST_CPW)

        @pl.when(c == 1)
        def _():
            pltpu.sync_copy(x_hbm.at[pl.ds(base, STRIPE)],
                            acc.at[pl.ds(base, STRIPE)])

        @pl.when(c != 1)
        def _():
            pltpu.sync_copy(z_hbm, acc.at[pl.ds(base, STRIPE)])

        plsc.subcore_barrier()

        for k in range(4):
            pltpu.async_copy(idx_hbm.at[cbase + k], ibufs[k], isems[k])
        pltpu.make_async_copy(idx_hbm.at[cbase], ibufs[0], isems[0]).wait()
        pltpu.async_copy(x_hbm.at[ibufs[0].at[0]], gbufs[0], gsems[0])

        def group(i, carry):
            for u in range(6):
                j = i * 6 + u
                b0 = u % NGB
                b1 = (u + 1) % NGB
                i0 = u % NIB
                i1 = (u + 1) % NIB
                i4 = (u + 4) % NIB
                pltpu.make_async_copy(
                    x_hbm.at[ibufs[i0].at[0]], gbufs[b0], gsems[b0]).wait()
                pltpu.async_copy(
                    gbufs[b0], acc.at[ibufs[i0].at[1]], ssems[b0], add=True)

                @pl.when(j >= 2)
                def _():
                    pltpu.make_async_copy(
                        gbufs[b1], acc.at[ibufs[i4].at[1]], ssems[b1]).wait()

                @pl.when(j + 4 < n)
                def _():
                    pltpu.async_copy(
                        idx_hbm.at[cbase + j + 4], ibufs[i4], isems[i4])

                @pl.when(j + 1 < n)
                def _():
                    pltpu.make_async_copy(
                        idx_hbm.at[cbase + j + 1], ibufs[i1], isems[i1]).wait()
                    pltpu.async_copy(
                        x_hbm.at[ibufs[i1].at[0]], gbufs[b1], gsems[b1])
            return carry

        lax.fori_loop(0, SLOW_CPW // 6, group, 0)

        @pl.when(jnp.logical_not(is_slow))
        def _():
            lax.fori_loop(SLOW_CPW // 6, FAST_CPW // 6, group, 0)

        pltpu.make_async_copy(gbufs[1], acc.at[ibufs[4].at[1]],
                              ssems[1]).wait()
        pltpu.make_async_copy(gbufs[2], acc.at[ibufs[5].at[1]],
                              ssems[2]).wait()

        plsc.subcore_barrier()

        pltpu.sync_copy(acc.at[pl.ds(base, STRIPE)],
                        out_hbm.at[c, pl.ds(base, STRIPE)])

    return agg_kernel(x_pad, idxp, zrows)


def _mm_block(a0_ref, a1_ref, w_ref, b_ref, o_ref):
    s = a0_ref[...] + a1_ref[...]
    o_ref[...] = (jnp.dot(s, w_ref[...], preferred_element_type=jnp.float32)
                  + b_ref[...])


def _final_linear(p0, p1, W, b):
    blk = 2000
    grid = (N_NODES // blk,)
    return pl.pallas_call(
        _mm_block,
        grid=grid,
        in_specs=[
            pl.BlockSpec((blk, D), lambda i: (i, 0)),
            pl.BlockSpec((blk, D), lambda i: (i, 0)),
            pl.BlockSpec((D, D), lambda i: (0, 0)),
            pl.BlockSpec((1, D), lambda i: (0, 0)),
        ],
        out_specs=pl.BlockSpec((blk, D), lambda i: (i, 0)),
        out_shape=jax.ShapeDtypeStruct((N_NODES, D), jnp.float32),
    )(p0, p1, W, b.reshape(1, D))


def kernel(x, edge_index, W, b):
    row = edge_index[0].astype(jnp.int32)
    col = edge_index[1].astype(jnp.int32)
    pad = E_PAD - N_EDGES
    colp = jnp.concatenate([col, jnp.zeros((pad,), jnp.int32)]
                           ).reshape(TOT_CHUNKS, 1, CS)
    dummy = N_NODES + (jnp.arange(pad, dtype=jnp.int32) % (N_PAD - N_NODES))
    rowp = jnp.concatenate([row, dummy]).reshape(TOT_CHUNKS, 1, CS)
    idxp = jnp.concatenate([colp, rowp], axis=1)
    x_pad = jnp.concatenate(
        [x, jnp.zeros((N_PAD - N_NODES, D), jnp.float32)])
    zrows = jnp.zeros((STRIPE, D), jnp.float32)
    parts = _sc_aggregate(x_pad, idxp, zrows)
    return _final_linear(parts[0, :N_NODES], parts[1, :N_NODES], W, b)


# speedup vs baseline: 1.0317x; 1.0317x over previous
import functools

import jax
import jax.numpy as jnp
from jax import lax
from jax.experimental import pallas as pl
from jax.experimental.pallas import tpu as pltpu
from jax.experimental.pallas import tpu_sc as plsc

N_NODES = 10000
N_EDGES = 320000
D = 128

NC = 2
NS = 16
NW = NC * NS
CS = 120
SLOW_C = 1
SLOW_CPW = 60
FAST_CPW = 108
NGB = 3
NIB = 6
TOT_CHUNKS = NS * (SLOW_CPW + FAST_CPW)
E_PAD = TOT_CHUNKS * CS
N_PAD = 10112
STRIPE = N_PAD // NS


def _sc_aggregate(x_pad, idxp, zrows):
    mesh = plsc.VectorSubcoreMesh(core_axis_name="c", subcore_axis_name="s",
                                  num_cores=NC, num_subcores=NS)

    @functools.partial(
        pl.kernel,
        out_type=jax.ShapeDtypeStruct((NC, N_PAD, D), jnp.float32),
        mesh=mesh,
        scratch_types=(
            [pltpu.VMEM((2, CS), jnp.int32) for _ in range(NIB)]
            + [pltpu.VMEM((CS, D), jnp.float32) for _ in range(NGB)]
            + [pltpu.SemaphoreType.DMA] * (NIB + 2 * NGB)
            + [pltpu.VMEM_SHARED((N_PAD, D), jnp.float32)]
        ),
    )
    def agg_kernel(x_hbm, idx_hbm, z_hbm, out_hbm, *scr):
        ibufs = scr[:NIB]
        gbufs = scr[NIB:NIB + NGB]
        isems = scr[NIB + NGB:2 * NIB + NGB]
        gsems = scr[2 * NIB + NGB:2 * NIB + 2 * NGB]
        ssems = scr[2 * NIB + 2 * NGB:2 * NIB + 3 * NGB]
        acc = scr[2 * NIB + 3 * NGB]
        c = lax.axis_index("c")
        s = lax.axis_index("s")
        base = s * STRIPE
        is_slow = c == SLOW_C
        cbase = jnp.where(is_slow, s * SLOW_CPW,
                          NS * SLOW_CPW + s * FAST_CPW)
        n = jnp.where(is_slow, SLOW_CPW, FAST_CPW)

        @pl.when(c == 1)
        def _():
            pltpu.sync_copy(x_hbm.at[pl.ds(base, STRIPE)],
                            acc.at[pl.ds(base, STRIPE)])

        @pl.when(c != 1)
        def _():
            pltpu.sync_copy(z_hbm, acc.at[pl.ds(base, STRIPE)])

        plsc.subcore_barrier()

        for k in range(4):
            pltpu.async_copy(idx_hbm.at[cbase + k], ibufs[k], isems[k])
        pltpu.make_async_copy(idx_hbm.at[cbase], ibufs[0], isems[0]).wait()
        pltpu.async_copy(x_hbm.at[ibufs[0].at[0]], gbufs[0], gsems[0])

        def group(i, carry):
            for u in range(6):
                j = i * 6 + u
                b0 = u % NGB
                b1 = (u + 1) % NGB
                i0 = u % NIB
                i1 = (u + 1) % NIB
                i4 = (u + 4) % NIB
                pltpu.make_async_copy(
                    x_hbm.at[ibufs[i0].at[0]], gbufs[b0], gsems[b0]).wait()
                pltpu.async_copy(
                    gbufs[b0], acc.at[ibufs[i0].at[1]], ssems[b0], add=True)

                @pl.when(j >= 2)
                def _():
                    pltpu.make_async_copy(
                        gbufs[b1], acc.at[ibufs[i4].at[1]], ssems[b1]).wait()

                @pl.when(j + 4 < n)
                def _():
                    pltpu.async_copy(
                        idx_hbm.at[cbase + j + 4], ibufs[i4], isems[i4])

                @pl.when(j + 1 < n)
                def _():
                    pltpu.make_async_copy(
                        idx_hbm.at[cbase + j + 1], ibufs[i1], isems[i1]).wait()
                    pltpu.async_copy(
                        x_hbm.at[ibufs[i1].at[0]], gbufs[b1], gsems[b1])
            return carry

        lax.fori_loop(0, SLOW_CPW // 6, group, 0)

        @pl.when(jnp.logical_not(is_slow))
        def _():
            lax.fori_loop(SLOW_CPW // 6, FAST_CPW // 6, group, 0)

        pltpu.make_async_copy(gbufs[1], acc.at[ibufs[4].at[1]],
                              ssems[1]).wait()
        pltpu.make_async_copy(gbufs[2], acc.at[ibufs[5].at[1]],
                              ssems[2]).wait()

        plsc.subcore_barrier()

        pltpu.sync_copy(acc.at[pl.ds(base, STRIPE)],
                        out_hbm.at[c, pl.ds(base, STRIPE)])

    return agg_kernel(x_pad, idxp, zrows)


def _mm_block(a0_ref, a1_ref, w_ref, b_ref, o_ref):
    s = a0_ref[...] + a1_ref[...]
    o_ref[...] = (jnp.dot(s, w_ref[...], preferred_element_type=jnp.float32)
                  + b_ref[...])


def _final_linear(p0, p1, W, b):
    blk = 2000
    grid = (N_NODES // blk,)
    return pl.pallas_call(
        _mm_block,
        grid=grid,
        in_specs=[
            pl.BlockSpec((blk, D), lambda i: (i, 0)),
            pl.BlockSpec((blk, D), lambda i: (i, 0)),
            pl.BlockSpec((D, D), lambda i: (0, 0)),
            pl.BlockSpec((1, D), lambda i: (0, 0)),
        ],
        out_specs=pl.BlockSpec((blk, D), lambda i: (i, 0)),
        out_shape=jax.ShapeDtypeStruct((N_NODES, D), jnp.float32),
    )(p0, p1, W, b.reshape(1, D))


def kernel(x, edge_index, W, b):
    row = edge_index[0].astype(jnp.int32)
    col = edge_index[1].astype(jnp.int32)
    pad = E_PAD - N_EDGES
    colp = jnp.concatenate([col, jnp.zeros((pad,), jnp.int32)]
                           ).reshape(TOT_CHUNKS, 1, CS)
    dummy = N_NODES + (jnp.arange(pad, dtype=jnp.int32) % (N_PAD - N_NODES))
    rowp = jnp.concatenate([row, dummy]).reshape(TOT_CHUNKS, 1, CS)
    idxp = jnp.concatenate([colp, rowp], axis=1)
    x_pad = jnp.concatenate(
        [x, jnp.zeros((N_PAD - N_NODES, D), jnp.float32)])
    zrows = jnp.zeros((STRIPE, D), jnp.float32)
    parts = _sc_aggregate(x_pad, idxp, zrows)
    return _final_linear(parts[0, :N_NODES], parts[1, :N_NODES], W, b)


# speedup vs baseline: 1.2132x; 1.1760x over previous
import functools

import jax
import jax.numpy as jnp
from jax import lax
from jax.experimental import pallas as pl
from jax.experimental.pallas import tpu as pltpu
from jax.experimental.pallas import tpu_sc as plsc

N_NODES = 10000
N_EDGES = 320000
D = 128

NC = 2
NS = 16
NW = NC * NS
CS = 120
SLOW_C = 1
SLOW_CPW = 84
FAST_CPW = 84
NGB = 3
NIB = 6
TOT_CHUNKS = NS * (SLOW_CPW + FAST_CPW)
E_PAD = TOT_CHUNKS * CS
N_PAD = 10112
STRIPE = N_PAD // NS


def _sc_aggregate(x_pad, idxp, zrows):
    mesh = plsc.VectorSubcoreMesh(core_axis_name="c", subcore_axis_name="s",
                                  num_cores=NC, num_subcores=NS)

    @functools.partial(
        pl.kernel,
        out_type=jax.ShapeDtypeStruct((NC, N_PAD, D), jnp.float32),
        mesh=mesh,
        scratch_types=(
            [pltpu.VMEM((2, CS), jnp.int32) for _ in range(NIB)]
            + [pltpu.VMEM((CS, D), jnp.float32) for _ in range(NGB)]
            + [pltpu.SemaphoreType.DMA] * (NIB + 2 * NGB)
            + [pltpu.VMEM_SHARED((N_PAD, D), jnp.float32)]
        ),
    )
    def agg_kernel(x_hbm, idx_hbm, z_hbm, out_hbm, *scr):
        ibufs = scr[:NIB]
        gbufs = scr[NIB:NIB + NGB]
        isems = scr[NIB + NGB:2 * NIB + NGB]
        gsems = scr[2 * NIB + NGB:2 * NIB + 2 * NGB]
        ssems = scr[2 * NIB + 2 * NGB:2 * NIB + 3 * NGB]
        acc = scr[2 * NIB + 3 * NGB]
        c = lax.axis_index("c")
        s = lax.axis_index("s")
        base = s * STRIPE
        is_slow = c == SLOW_C
        cbase = jnp.where(is_slow, s * SLOW_CPW,
                          NS * SLOW_CPW + s * FAST_CPW)
        n = jnp.where(is_slow, SLOW_CPW, FAST_CPW)

        @pl.when(c == 1)
        def _():
            pltpu.sync_copy(x_hbm.at[pl.ds(base, STRIPE)],
                            acc.at[pl.ds(base, STRIPE)])

        @pl.when(c != 1)
        def _():
            pltpu.sync_copy(z_hbm, acc.at[pl.ds(base, STRIPE)])

        plsc.subcore_barrier()

        for k in range(4):
            pltpu.async_copy(idx_hbm.at[cbase + k], ibufs[k], isems[k])
        pltpu.make_async_copy(idx_hbm.at[cbase], ibufs[0], isems[0]).wait()
        pltpu.async_copy(x_hbm.at[ibufs[0].at[0]], gbufs[0], gsems[0])

        def group(i, carry):
            for u in range(6):
                j = i * 6 + u
                b0 = u % NGB
                b1 = (u + 1) % NGB
                i0 = u % NIB
                i1 = (u + 1) % NIB
                i4 = (u + 4) % NIB
                @pl.when(j >= 2)
                def _():
                    pltpu.make_async_copy(
                        gbufs[b1], acc.at[ibufs[i4].at[1]], ssems[b1]).wait()

                @pl.when(j + 4 < n)
                def _():
                    pltpu.async_copy(
                        idx_hbm.at[cbase + j + 4], ibufs[i4], isems[i4])

                @pl.when(j + 1 < n)
                def _():
                    pltpu.make_async_copy(
                        idx_hbm.at[cbase + j + 1], ibufs[i1], isems[i1]).wait()
                    pltpu.async_copy(
                        x_hbm.at[ibufs[i1].at[0]], gbufs[b1], gsems[b1])

                pltpu.make_async_copy(
                    x_hbm.at[ibufs[i0].at[0]], gbufs[b0], gsems[b0]).wait()
                pltpu.async_copy(
                    gbufs[b0], acc.at[ibufs[i0].at[1]], ssems[b0], add=True)
            return carry

        lax.fori_loop(0, SLOW_CPW // 6, group, 0)

        @pl.when(jnp.logical_not(is_slow))
        def _():
            lax.fori_loop(SLOW_CPW // 6, FAST_CPW // 6, group, 0)

        pltpu.make_async_copy(gbufs[1], acc.at[ibufs[4].at[1]],
                              ssems[1]).wait()
        pltpu.make_async_copy(gbufs[2], acc.at[ibufs[5].at[1]],
                              ssems[2]).wait()

        plsc.subcore_barrier()

        pltpu.sync_copy(acc.at[pl.ds(base, STRIPE)],
                        out_hbm.at[c, pl.ds(base, STRIPE)])

    return agg_kernel(x_pad, idxp, zrows)


def _mm_block(a0_ref, a1_ref, w_ref, b_ref, o_ref):
    s = a0_ref[...] + a1_ref[...]
    o_ref[...] = (jnp.dot(s, w_ref[...], preferred_element_type=jnp.float32)
                  + b_ref[...])


def _final_linear(p0, p1, W, b):
    blk = 2000
    grid = (N_NODES // blk,)
    return pl.pallas_call(
        _mm_block,
        grid=grid,
        in_specs=[
            pl.BlockSpec((blk, D), lambda i: (i, 0)),
            pl.BlockSpec((blk, D), lambda i: (i, 0)),
            pl.BlockSpec((D, D), lambda i: (0, 0)),
            pl.BlockSpec((1, D), lambda i: (0, 0)),
        ],
        out_specs=pl.BlockSpec((blk, D), lambda i: (i, 0)),
        out_shape=jax.ShapeDtypeStruct((N_NODES, D), jnp.float32),
    )(p0, p1, W, b.reshape(1, D))


def kernel(x, edge_index, W, b):
    row = edge_index[0].astype(jnp.int32)
    col = edge_index[1].astype(jnp.int32)
    pad = E_PAD - N_EDGES
    colp = jnp.concatenate([col, jnp.zeros((pad,), jnp.int32)]
                           ).reshape(TOT_CHUNKS, 1, CS)
    dummy = N_NODES + (jnp.arange(pad, dtype=jnp.int32) % (N_PAD - N_NODES))
    rowp = jnp.concatenate([row, dummy]).reshape(TOT_CHUNKS, 1, CS)
    idxp = jnp.concatenate([colp, rowp], axis=1)
    x_pad = jnp.concatenate(
        [x, jnp.zeros((N_PAD - N_NODES, D), jnp.float32)])
    zrows = jnp.zeros((STRIPE, D), jnp.float32)
    parts = _sc_aggregate(x_pad, idxp, zrows)
    return _final_linear(parts[0, :N_NODES], parts[1, :N_NODES], W, b)
